# baseline (speedup 1.0000x reference)
import functools

import jax
import jax.numpy as jnp
from jax import lax
from jax.experimental import pallas as pl
from jax.experimental.pallas import tpu as pltpu
from jax.experimental.pallas import tpu_sc as plsc

N = 10000
E = 160000
H = 256
HH = 128
EF = 16

NS = 16
CHUNK = 64
E_PAD = 163840
NCHUNKS = E_PAD // CHUNK
CPT = NCHUNKS // NS
DBLK = 40
NDBLK = N // DBLK
DITERS = -(-NDBLK // NS)
VEC = 16


def _zero_buf(buf):
    @pl.loop(0, CHUNK)
    def _(j):
        z = jnp.zeros((VEC,), jnp.float32)
        for k in range(HH // VEC):
            buf[j, pl.ds(k * VEC, VEC)] = z


def _sc_edge_kernel(x_lo, x_hi, e_lo, e_hi, src_ids, dst_ids, agg_lo, agg_hi,
                    src_v, dst_v, rows_v, e_v, acc, sem_i, sem_d, sem_g,
                    sem_e, sem_s):
    c = lax.axis_index("c")
    s = lax.axis_index("s")

    _zero_buf(rows_v.at[0])

    @pl.loop(0, DITERS)
    def _(k):
        blk = k * NS + s

        @pl.when(blk < NDBLK)
        def _():
            pltpu.sync_copy(rows_v.at[0, pl.ds(0, DBLK)],
                            acc.at[pl.ds(blk * DBLK, DBLK)])

    plsc.subcore_barrier()

    def edge_loop(x_tab, e_tab):
        def issue_idx(k, q8):
            ebase = (k * NS + s) * CHUNK
            pltpu.async_copy(src_ids.at[pl.ds(ebase, CHUNK)],
                             src_v.at[q8 % 4], sem_i.at[q8 % 4])
            pltpu.async_copy(dst_ids.at[pl.ds(ebase, CHUNK)], dst_v.at[q8],
                             sem_d.at[q8])

        def wait_src(q8):
            pltpu.make_async_copy(src_ids.at[pl.ds(0, CHUNK)],
                                  src_v.at[q8 % 4], sem_i.at[q8 % 4]).wait()

        def wait_dst(q8):
            pltpu.make_async_copy(dst_ids.at[pl.ds(0, CHUNK)], dst_v.at[q8],
                                  sem_d.at[q8]).wait()

        def issue_gather(q8, q4):
            pltpu.async_copy(x_tab.at[src_v.at[q8 % 4]], rows_v.at[q4],
                             sem_g.at[q4])

        def wait_gather(q8, q4):
            pltpu.make_async_copy(x_tab.at[src_v.at[q8 % 4]], rows_v.at[q4],
                                  sem_g.at[q4]).wait()

        def issue_e(k, b2):
            ebase = (k * NS + s) * CHUNK
            pltpu.async_copy(e_tab.at[pl.ds(ebase, CHUNK)], e_v.at[b2],
                             sem_e.at[b2])

        def wait_e(b2):
            pltpu.make_async_copy(e_tab.at[pl.ds(0, CHUNK)], e_v.at[b2],
                                  sem_e.at[b2]).wait()

        def issue_scatter(q8, q4):
            pltpu.async_copy(rows_v.at[q4], acc.at[dst_v.at[q8]], sem_s.at[q4],
                             add=True)

        def wait_scatter(q8, q4):
            pltpu.make_async_copy(rows_v.at[q4], acc.at[dst_v.at[q8]],
                                  sem_s.at[q4]).wait()

        for k0 in range(4):
            issue_idx(k0, k0)
        for k0 in range(2):
            wait_src(k0)
            issue_gather(k0, k0)
            issue_e(k0, k0)

        @pl.loop(0, CPT // 8)
        def _(ko):
            for bb in range(8):
                k = ko * 8 + bb
                q4 = bb % 4
                b2 = bb % 2
                f4 = (bb + 2) % 4
                f8 = (bb + 2) % 8
                p8 = (bb + 6) % 8

                wait_gather(bb, q4)
                wait_e(b2)

                @pl.when(k >= 2)
                def _():
                    wait_scatter(p8, f4)

                @pl.when(k < CPT - 2)
                def _():
                    wait_src(f8)
                    issue_gather(f8, f4)

                @pl.loop(0, CHUNK, unroll=2)
                def _(j):
                    for t in range(HH // VEC):
                        sl = pl.ds(t * VEC, VEC)
                        rows_v[q4, j, sl] = jnp.maximum(
                            rows_v[q4, j, sl] + e_v[b2, j, sl], 0.0)

                @pl.when(k < CPT - 2)
                def _():
                    issue_e(k + 2, b2)

                wait_dst(bb)
                issue_scatter(bb, q4)

                @pl.when(k < CPT - 4)
                def _():
                    issue_idx(k + 4, (bb + 4) % 8)

        wait_scatter((CPT - 2) % 8, (CPT - 2) % 4)
        wait_scatter((CPT - 1) % 8, (CPT - 1) % 4)

    @pl.when(c == 0)
    def _():
        edge_loop(x_lo, e_lo)

    @pl.when(c == 1)
    def _():
        edge_loop(x_hi, e_hi)

    plsc.subcore_barrier()

    def dump(out):
        @pl.loop(0, DITERS)
        def _(k):
            blk = k * NS + s

            @pl.when(blk < NDBLK)
            def _():
                pltpu.sync_copy(acc.at[pl.ds(blk * DBLK, DBLK)],
                                rows_v.at[0, pl.ds(0, DBLK)])
                pltpu.sync_copy(rows_v.at[0, pl.ds(0, DBLK)],
                                out.at[pl.ds(blk * DBLK, DBLK)])

    @pl.when(c == 0)
    def _():
        dump(agg_lo)

    @pl.when(c == 1)
    def _():
        dump(agg_hi)


_sc_edge_phase = functools.partial(
    pl.kernel,
    out_type=(jax.ShapeDtypeStruct((N, HH), jnp.float32),
              jax.ShapeDtypeStruct((N, HH), jnp.float32)),
    mesh=plsc.VectorSubcoreMesh(core_axis_name="c", subcore_axis_name="s",
                                num_cores=2, num_subcores=NS),
    scratch_types=[
        pltpu.VMEM((4, CHUNK), jnp.int32),
        pltpu.VMEM((8, CHUNK), jnp.int32),
        pltpu.VMEM((4, CHUNK, HH), jnp.float32),
        pltpu.VMEM((2, CHUNK, HH), jnp.float32),
        pltpu.VMEM_SHARED((N, HH), jnp.float32),
        pltpu.SemaphoreType.DMA((4,)),
        pltpu.SemaphoreType.DMA((8,)),
        pltpu.SemaphoreType.DMA((4,)),
        pltpu.SemaphoreType.DMA((2,)),
        pltpu.SemaphoreType.DMA((4,)),
    ],
)(_sc_edge_kernel)


BE = 2048


def _edge_mlp_kernel(ef_ref, w_ref, b_ref, elo_ref, ehi_ref):
    e = jnp.dot(ef_ref[...], w_ref[...],
                preferred_element_type=jnp.float32) + b_ref[...]
    rid = lax.broadcasted_iota(jnp.int32, (BE, H), 0) + pl.program_id(0) * BE
    e = jnp.where(rid < E, e, -1e30)
    elo_ref[...] = e[:, :HH]
    ehi_ref[...] = e[:, HH:]


def _edge_mlp(edge_feat, edge_W, edge_b):
    return pl.pallas_call(
        _edge_mlp_kernel,
        grid=(E_PAD // BE,),
        in_specs=[
            pl.BlockSpec((BE, EF), lambda j: (j, 0)),
            pl.BlockSpec((EF, H), lambda j: (0, 0)),
            pl.BlockSpec((1, H), lambda j: (0, 0)),
        ],
        out_specs=[
            pl.BlockSpec((BE, HH), lambda j: (j, 0)),
            pl.BlockSpec((BE, HH), lambda j: (j, 0)),
        ],
        out_shape=[
            jax.ShapeDtypeStruct((E_PAD, HH), jnp.float32),
            jax.ShapeDtypeStruct((E_PAD, HH), jnp.float32),
        ],
    )(edge_feat, edge_W, edge_b)


BN = 1000
NB = N // BN


def _layer_tc_kernel(xlo_ref, xhi_ref, alo_ref, ahi_ref, w1_ref, b1_ref,
                     w2_ref, b2_ref, g_ref, bt_ref, nxlo_ref, nxhi_ref,
                     u_scr, sum_scr, sq_scr, sc_scr, sh_scr):
    p = pl.program_id(0)
    j = pl.program_id(1)

    @pl.when(p == 0)
    def _():
        h = jnp.concatenate(
            [xlo_ref[...] + alo_ref[...], xhi_ref[...] + ahi_ref[...]], axis=1)
        t = jnp.maximum(
            jnp.dot(h, w1_ref[...], preferred_element_type=jnp.float32)
            + b1_ref[...], 0.0)
        u = jnp.dot(t, w2_ref[...],
                    preferred_element_type=jnp.float32) + b2_ref[...]
        u_scr[pl.ds(j * BN, BN), :] = u
        su = jnp.sum(u, axis=0, keepdims=True)
        sq = jnp.sum(u * u, axis=0, keepdims=True)

        @pl.when(j == 0)
        def _():
            sum_scr[...] = su
            sq_scr[...] = sq

        @pl.when(j > 0)
        def _():
            sum_scr[...] += su
            sq_scr[...] += sq

    @pl.when(p == 1)
    def _():
        @pl.when(j == 0)
        def _():
            mean = sum_scr[...] * (1.0 / N)
            var = sq_scr[...] * (1.0 / N) - mean * mean
            inv = lax.rsqrt(var + 1e-5)
            scale = g_ref[...] * inv
            sc_scr[...] = scale
            sh_scr[...] = bt_ref[...] - mean * scale

        u = u_scr[pl.ds(j * BN, BN), :]
        y = jnp.maximum(u * sc_scr[...] + sh_scr[...], 0.0)
        nxlo_ref[...] = y[:, :HH] + xlo_ref[...]
        nxhi_ref[...] = y[:, HH:] + xhi_ref[...]


def _layer_tc(x_lo, x_hi, agg_lo, agg_hi, w1, b1, w2, b2, g, bt):
    node_spec = pl.BlockSpec((BN, HH), lambda p, j: (j, 0))
    full_spec = pl.BlockSpec((H, H), lambda p, j: (0, 0))
    row_spec = pl.BlockSpec((1, H), lambda p, j: (0, 0))
    return pl.pallas_call(
        _layer_tc_kernel,
        grid=(2, NB),
        in_specs=[node_spec, node_spec, node_spec, node_spec,
                  full_spec, row_spec, full_spec, row_spec,
                  row_spec, row_spec],
        out_specs=[node_spec, node_spec],
        out_shape=[
            jax.ShapeDtypeStruct((N, HH), jnp.float32),
            jax.ShapeDtypeStruct((N, HH), jnp.float32),
        ],
        scratch_shapes=[
            pltpu.VMEM((N, H), jnp.float32),
            pltpu.VMEM((1, H), jnp.float32),
            pltpu.VMEM((1, H), jnp.float32),
            pltpu.VMEM((1, H), jnp.float32),
            pltpu.VMEM((1, H), jnp.float32),
        ],
    )(x_lo, x_hi, agg_lo, agg_hi, w1, b1, w2, b2, g, bt)


def kernel(node_feat, edge_index, edge_feat, edge_W, edge_b, W1, b1, W2, b2,
           gamma, beta):
    pad = jnp.zeros((E_PAD - E,), jnp.int32)
    src = jnp.concatenate([edge_index[0].astype(jnp.int32), pad])
    dst = jnp.concatenate([edge_index[1].astype(jnp.int32), pad])
    ef_pad = jnp.concatenate(
        [edge_feat, jnp.zeros((E_PAD - E, EF), jnp.float32)])

    e_lo, e_hi = _edge_mlp(ef_pad, edge_W, edge_b.reshape(1, H))

    x_lo = node_feat[:, :HH]
    x_hi = node_feat[:, HH:]
    for i in range(W1.shape[0]):
        agg_lo, agg_hi = _sc_edge_phase(x_lo, x_hi, e_lo, e_hi, src, dst)
        x_lo, x_hi = _layer_tc(x_lo, x_hi, agg_lo, agg_hi,
                               W1[i], b1[i].reshape(1, H),
                               W2[i], b2[i].reshape(1, H),
                               gamma[i].reshape(1, H), beta[i].reshape(1, H))
    return jnp.concatenate([x_lo, x_hi], axis=1)


# speedup vs baseline: 1.1347x; 1.1347x over previous
import functools

import jax
import jax.numpy as jnp
from jax import lax
from jax.experimental import pallas as pl
from jax.experimental.pallas import tpu as pltpu
from jax.experimental.pallas import tpu_sc as plsc

N = 10000
E = 160000
H = 256
HH = 128
EF = 16

NS = 16
CHUNK = 64
E_PAD = 163840
NCHUNKS = E_PAD // CHUNK
CPT = NCHUNKS // NS
DBLK = 40
NDBLK = N // DBLK
DITERS = -(-NDBLK // NS)
VEC = 16


def _zero_buf(buf):
    @pl.loop(0, CHUNK)
    def _(j):
        z = jnp.zeros((VEC,), jnp.float32)
        for k in range(HH // VEC):
            buf[j, pl.ds(k * VEC, VEC)] = z


def _sc_edge_kernel(x_lo, x_hi, e_lo, e_hi, src_ids, dst_ids, agg_lo, agg_hi,
                    src_v, dst_v, rows_v, e_v, acc, si0, si1, sd0, sd1):
    c = lax.axis_index("c")
    s = lax.axis_index("s")

    _zero_buf(rows_v.at[0])

    @pl.loop(0, DITERS)
    def _(k):
        blk = k * NS + s

        @pl.when(blk < NDBLK)
        def _():
            pltpu.sync_copy(rows_v.at[0, pl.ds(0, DBLK)],
                            acc.at[pl.ds(blk * DBLK, DBLK)])

    plsc.subcore_barrier()

    sems_i = (si0, si1)
    sems_d = (sd0, sd1)

    def edge_loop(x_tab, e_tab):
        def issue_idx(k, b):
            ebase = (k * NS + s) * CHUNK
            pltpu.async_copy(src_ids.at[pl.ds(ebase, CHUNK)], src_v.at[b],
                             sems_i[b])
            pltpu.async_copy(dst_ids.at[pl.ds(ebase, CHUNK)], dst_v.at[b],
                             sems_i[b])

        def wait_idx(b):
            pltpu.make_async_copy(src_ids.at[pl.ds(0, CHUNK)], src_v.at[b],
                                  sems_i[b]).wait()
            pltpu.make_async_copy(dst_ids.at[pl.ds(0, CHUNK)], dst_v.at[b],
                                  sems_i[b]).wait()

        def issue_data(k, b):
            ebase = (k * NS + s) * CHUNK
            pltpu.async_copy(x_tab.at[src_v.at[b]], rows_v.at[b], sems_d[b])
            pltpu.async_copy(e_tab.at[pl.ds(ebase, CHUNK)], e_v.at[b],
                             sems_d[b])

        def wait_data(b):
            pltpu.make_async_copy(x_tab.at[src_v.at[b]], rows_v.at[b],
                                  sems_d[b]).wait()
            pltpu.make_async_copy(e_tab.at[pl.ds(0, CHUNK)], e_v.at[b],
                                  sems_d[b]).wait()

        issue_idx(0, 0)
        issue_idx(1, 1)
        wait_idx(0)
        issue_data(0, 0)

        @pl.loop(0, CPT // 2)
        def _(ko):
            for b in range(2):
                k = ko * 2 + b
                ob = 1 - b
                wait_data(b)

                @pl.when(k < CPT - 1)
                def _():
                    wait_idx(ob)
                    issue_data(k + 1, ob)

                @plsc.parallel_loop(0, CHUNK, unroll=2)
                def _(j):
                    for t in range(HH // VEC):
                        sl = pl.ds(t * VEC, VEC)
                        rows_v[b, j, sl] = jnp.maximum(
                            rows_v[b, j, sl] + e_v[b, j, sl], 0.0)

                pltpu.sync_copy(rows_v.at[b], acc.at[dst_v.at[b]], add=True)

                @pl.when(k < CPT - 2)
                def _():
                    issue_idx(k + 2, b)

    @pl.when(c == 0)
    def _():
        edge_loop(x_lo, e_lo)

    @pl.when(c == 1)
    def _():
        edge_loop(x_hi, e_hi)

    plsc.subcore_barrier()

    def dump(out):
        @pl.loop(0, DITERS)
        def _(k):
            blk = k * NS + s

            @pl.when(blk < NDBLK)
            def _():
                pltpu.sync_copy(acc.at[pl.ds(blk * DBLK, DBLK)],
                                rows_v.at[0, pl.ds(0, DBLK)])
                pltpu.sync_copy(rows_v.at[0, pl.ds(0, DBLK)],
                                out.at[pl.ds(blk * DBLK, DBLK)])

    @pl.when(c == 0)
    def _():
        dump(agg_lo)

    @pl.when(c == 1)
    def _():
        dump(agg_hi)


_sc_edge_phase = functools.partial(
    pl.kernel,
    out_type=(jax.ShapeDtypeStruct((N, HH), jnp.float32),
              jax.ShapeDtypeStruct((N, HH), jnp.float32)),
    mesh=plsc.VectorSubcoreMesh(core_axis_name="c", subcore_axis_name="s",
                                num_cores=2, num_subcores=NS),
    scratch_types=[
        pltpu.VMEM((2, CHUNK), jnp.int32),
        pltpu.VMEM((2, CHUNK), jnp.int32),
        pltpu.VMEM((2, CHUNK, HH), jnp.float32),
        pltpu.VMEM((2, CHUNK, HH), jnp.float32),
        pltpu.VMEM_SHARED((N, HH), jnp.float32),
        pltpu.SemaphoreType.DMA,
        pltpu.SemaphoreType.DMA,
        pltpu.SemaphoreType.DMA,
        pltpu.SemaphoreType.DMA,
    ],
)(_sc_edge_kernel)


BE = 2048


def _edge_mlp_kernel(ef_ref, w_ref, b_ref, elo_ref, ehi_ref):
    e = jnp.dot(ef_ref[...], w_ref[...],
                preferred_element_type=jnp.float32) + b_ref[...]
    rid = lax.broadcasted_iota(jnp.int32, (BE, H), 0) + pl.program_id(0) * BE
    e = jnp.where(rid < E, e, -1e30)
    elo_ref[...] = e[:, :HH]
    ehi_ref[...] = e[:, HH:]


def _edge_mlp(edge_feat, edge_W, edge_b):
    return pl.pallas_call(
        _edge_mlp_kernel,
        grid=(E_PAD // BE,),
        in_specs=[
            pl.BlockSpec((BE, EF), lambda j: (j, 0)),
            pl.BlockSpec((EF, H), lambda j: (0, 0)),
            pl.BlockSpec((1, H), lambda j: (0, 0)),
        ],
        out_specs=[
            pl.BlockSpec((BE, HH), lambda j: (j, 0)),
            pl.BlockSpec((BE, HH), lambda j: (j, 0)),
        ],
        out_shape=[
            jax.ShapeDtypeStruct((E_PAD, HH), jnp.float32),
            jax.ShapeDtypeStruct((E_PAD, HH), jnp.float32),
        ],
    )(edge_feat, edge_W, edge_b)


BN = 1000
NB = N // BN


def _layer_tc_kernel(xlo_ref, xhi_ref, alo_ref, ahi_ref, w1_ref, b1_ref,
                     w2_ref, b2_ref, g_ref, bt_ref, nxlo_ref, nxhi_ref,
                     u_scr, sum_scr, sq_scr, sc_scr, sh_scr):
    p = pl.program_id(0)
    j = pl.program_id(1)

    @pl.when(p == 0)
    def _():
        h = jnp.concatenate(
            [xlo_ref[...] + alo_ref[...], xhi_ref[...] + ahi_ref[...]], axis=1)
        t = jnp.maximum(
            jnp.dot(h, w1_ref[...], preferred_element_type=jnp.float32)
            + b1_ref[...], 0.0)
        u = jnp.dot(t, w2_ref[...],
                    preferred_element_type=jnp.float32) + b2_ref[...]
        u_scr[pl.ds(j * BN, BN), :] = u
        su = jnp.sum(u, axis=0, keepdims=True)
        sq = jnp.sum(u * u, axis=0, keepdims=True)

        @pl.when(j == 0)
        def _():
            sum_scr[...] = su
            sq_scr[...] = sq

        @pl.when(j > 0)
        def _():
            sum_scr[...] += su
            sq_scr[...] += sq

    @pl.when(p == 1)
    def _():
        @pl.when(j == 0)
        def _():
            mean = sum_scr[...] * (1.0 / N)
            var = sq_scr[...] * (1.0 / N) - mean * mean
            inv = lax.rsqrt(var + 1e-5)
            scale = g_ref[...] * inv
            sc_scr[...] = scale
            sh_scr[...] = bt_ref[...] - mean * scale

        u = u_scr[pl.ds(j * BN, BN), :]
        y = jnp.maximum(u * sc_scr[...] + sh_scr[...], 0.0)
        nxlo_ref[...] = y[:, :HH] + xlo_ref[...]
        nxhi_ref[...] = y[:, HH:] + xhi_ref[...]


def _layer_tc(x_lo, x_hi, agg_lo, agg_hi, w1, b1, w2, b2, g, bt):
    node_spec = pl.BlockSpec((BN, HH), lambda p, j: (j, 0))
    full_spec = pl.BlockSpec((H, H), lambda p, j: (0, 0))
    row_spec = pl.BlockSpec((1, H), lambda p, j: (0, 0))
    return pl.pallas_call(
        _layer_tc_kernel,
        grid=(2, NB),
        in_specs=[node_spec, node_spec, node_spec, node_spec,
                  full_spec, row_spec, full_spec, row_spec,
                  row_spec, row_spec],
        out_specs=[node_spec, node_spec],
        out_shape=[
            jax.ShapeDtypeStruct((N, HH), jnp.float32),
            jax.ShapeDtypeStruct((N, HH), jnp.float32),
        ],
        scratch_shapes=[
            pltpu.VMEM((N, H), jnp.float32),
            pltpu.VMEM((1, H), jnp.float32),
            pltpu.VMEM((1, H), jnp.float32),
            pltpu.VMEM((1, H), jnp.float32),
            pltpu.VMEM((1, H), jnp.float32),
        ],
    )(x_lo, x_hi, agg_lo, agg_hi, w1, b1, w2, b2, g, bt)


def kernel(node_feat, edge_index, edge_feat, edge_W, edge_b, W1, b1, W2, b2,
           gamma, beta):
    pad = jnp.zeros((E_PAD - E,), jnp.int32)
    src = jnp.concatenate([edge_index[0].astype(jnp.int32), pad])
    dst = jnp.concatenate([edge_index[1].astype(jnp.int32), pad])
    ef_pad = jnp.concatenate(
        [edge_feat, jnp.zeros((E_PAD - E, EF), jnp.float32)])

    e_lo, e_hi = _edge_mlp(ef_pad, edge_W, edge_b.reshape(1, H))

    x_lo = node_feat[:, :HH]
    x_hi = node_feat[:, HH:]
    for i in range(W1.shape[0]):
        agg_lo, agg_hi = _sc_edge_phase(x_lo, x_hi, e_lo, e_hi, src, dst)
        x_lo, x_hi = _layer_tc(x_lo, x_hi, agg_lo, agg_hi,
                               W1[i], b1[i].reshape(1, H),
                               W2[i], b2[i].reshape(1, H),
                               gamma[i].reshape(1, H), beta[i].reshape(1, H))
    return jnp.concatenate([x_lo, x_hi], axis=1)


# speedup vs baseline: 1.1429x; 1.0072x over previous
import functools

import jax
import jax.numpy as jnp
from jax import lax
from jax.experimental import pallas as pl
from jax.experimental.pallas import tpu as pltpu
from jax.experimental.pallas import tpu_sc as plsc

N = 10000
E = 160000
H = 256
HH = 128
EF = 16

NS = 16
CHUNK = 64
E_PAD = 163840
NCHUNKS = E_PAD // CHUNK
CPT = NCHUNKS // NS
DBLK = 40
NDBLK = N // DBLK
DITERS = -(-NDBLK // NS)
VEC = 16


def _zero_buf(buf):
    @pl.loop(0, CHUNK)
    def _(j):
        z = jnp.zeros((VEC,), jnp.float32)
        for k in range(HH // VEC):
            buf[j, pl.ds(k * VEC, VEC)] = z


def _sc_edge_kernel(x_lo, x_hi, e_lo, e_hi, src_ids, dst_ids, agg_lo, agg_hi,
                    src_v, dst_v, rows_v, e_v, acc, si0, si1, sd0, sd1):
    c = lax.axis_index("c")
    s = lax.axis_index("s")

    _zero_buf(rows_v.at[0])

    @pl.loop(0, DITERS)
    def _(k):
        blk = k * NS + s

        @pl.when(blk < NDBLK)
        def _():
            pltpu.sync_copy(rows_v.at[0, pl.ds(0, DBLK)],
                            acc.at[pl.ds(blk * DBLK, DBLK)])

    plsc.subcore_barrier()

    sems_i = (si0, si1)
    sems_d = (sd0, sd1)

    def edge_loop(x_tab, e_tab):
        def issue_idx(k, b):
            ebase = (k * NS + s) * CHUNK
            pltpu.async_copy(src_ids.at[pl.ds(ebase, CHUNK)], src_v.at[b],
                             sems_i[b])
            pltpu.async_copy(dst_ids.at[pl.ds(ebase, CHUNK)], dst_v.at[b],
                             sems_i[b])

        def wait_idx(b):
            pltpu.make_async_copy(src_ids.at[pl.ds(0, CHUNK)], src_v.at[b],
                                  sems_i[b]).wait()
            pltpu.make_async_copy(dst_ids.at[pl.ds(0, CHUNK)], dst_v.at[b],
                                  sems_i[b]).wait()

        def issue_data(k, b):
            ebase = (k * NS + s) * CHUNK
            pltpu.async_copy(x_tab.at[src_v.at[b]], rows_v.at[b], sems_d[b])
            pltpu.async_copy(e_tab.at[pl.ds(ebase, CHUNK)], e_v.at[b],
                             sems_d[b])

        def wait_data(b):
            pltpu.make_async_copy(x_tab.at[src_v.at[b]], rows_v.at[b],
                                  sems_d[b]).wait()
            pltpu.make_async_copy(e_tab.at[pl.ds(0, CHUNK)], e_v.at[b],
                                  sems_d[b]).wait()

        issue_idx(0, 0)
        issue_idx(1, 1)
        wait_idx(0)
        issue_data(0, 0)

        @pl.loop(0, CPT // 2)
        def _(ko):
            for b in range(2):
                k = ko * 2 + b
                ob = 1 - b
                wait_data(b)

                @pl.when(k < CPT - 1)
                def _():
                    wait_idx(ob)
                    issue_data(k + 1, ob)

                @pl.loop(0, CHUNK)
                def _(j):
                    for t in range(HH // VEC):
                        sl = pl.ds(t * VEC, VEC)
                        rows_v[b, j, sl] = jnp.maximum(
                            rows_v[b, j, sl] + e_v[b, j, sl], 0.0)

                pltpu.sync_copy(rows_v.at[b], acc.at[dst_v.at[b]], add=True)

                @pl.when(k < CPT - 2)
                def _():
                    issue_idx(k + 2, b)

    @pl.when(c == 0)
    def _():
        edge_loop(x_lo, e_lo)

    @pl.when(c == 1)
    def _():
        edge_loop(x_hi, e_hi)

    plsc.subcore_barrier()

    def dump(out):
        @pl.loop(0, DITERS)
        def _(k):
            blk = k * NS + s

            @pl.when(blk < NDBLK)
            def _():
                pltpu.sync_copy(acc.at[pl.ds(blk * DBLK, DBLK)],
                                rows_v.at[0, pl.ds(0, DBLK)])
                pltpu.sync_copy(rows_v.at[0, pl.ds(0, DBLK)],
                                out.at[pl.ds(blk * DBLK, DBLK)])

    @pl.when(c == 0)
    def _():
        dump(agg_lo)

    @pl.when(c == 1)
    def _():
        dump(agg_hi)


_sc_edge_phase = functools.partial(
    pl.kernel,
    out_type=(jax.ShapeDtypeStruct((N, HH), jnp.float32),
              jax.ShapeDtypeStruct((N, HH), jnp.float32)),
    mesh=plsc.VectorSubcoreMesh(core_axis_name="c", subcore_axis_name="s",
                                num_cores=2, num_subcores=NS),
    scratch_types=[
        pltpu.VMEM((2, CHUNK), jnp.int32),
        pltpu.VMEM((2, CHUNK), jnp.int32),
        pltpu.VMEM((2, CHUNK, HH), jnp.float32),
        pltpu.VMEM((2, CHUNK, HH), jnp.float32),
        pltpu.VMEM_SHARED((N, HH), jnp.float32),
        pltpu.SemaphoreType.DMA,
        pltpu.SemaphoreType.DMA,
        pltpu.SemaphoreType.DMA,
        pltpu.SemaphoreType.DMA,
    ],
)(_sc_edge_kernel)


BE = 2048


def _edge_mlp_kernel(ef_ref, w_ref, b_ref, elo_ref, ehi_ref):
    e = jnp.dot(ef_ref[...], w_ref[...],
                preferred_element_type=jnp.float32) + b_ref[...]
    rid = lax.broadcasted_iota(jnp.int32, (BE, H), 0) + pl.program_id(0) * BE
    e = jnp.where(rid < E, e, -1e30)
    elo_ref[...] = e[:, :HH]
    ehi_ref[...] = e[:, HH:]


def _edge_mlp(edge_feat, edge_W, edge_b):
    return pl.pallas_call(
        _edge_mlp_kernel,
        grid=(E_PAD // BE,),
        in_specs=[
            pl.BlockSpec((BE, EF), lambda j: (j, 0)),
            pl.BlockSpec((EF, H), lambda j: (0, 0)),
            pl.BlockSpec((1, H), lambda j: (0, 0)),
        ],
        out_specs=[
            pl.BlockSpec((BE, HH), lambda j: (j, 0)),
            pl.BlockSpec((BE, HH), lambda j: (j, 0)),
        ],
        out_shape=[
            jax.ShapeDtypeStruct((E_PAD, HH), jnp.float32),
            jax.ShapeDtypeStruct((E_PAD, HH), jnp.float32),
        ],
    )(edge_feat, edge_W, edge_b)


BN = 1000
NB = N // BN


def _layer_tc_kernel(xlo_ref, xhi_ref, alo_ref, ahi_ref, w1_ref, b1_ref,
                     w2_ref, b2_ref, g_ref, bt_ref, nxlo_ref, nxhi_ref,
                     u_scr, sum_scr, sq_scr, sc_scr, sh_scr):
    p = pl.program_id(0)
    j = pl.program_id(1)

    @pl.when(p == 0)
    def _():
        h = jnp.concatenate(
            [xlo_ref[...] + alo_ref[...], xhi_ref[...] + ahi_ref[...]], axis=1)
        t = jnp.maximum(
            jnp.dot(h, w1_ref[...], preferred_element_type=jnp.float32)
            + b1_ref[...], 0.0)
        u = jnp.dot(t, w2_ref[...],
                    preferred_element_type=jnp.float32) + b2_ref[...]
        u_scr[pl.ds(j * BN, BN), :] = u
        su = jnp.sum(u, axis=0, keepdims=True)
        sq = jnp.sum(u * u, axis=0, keepdims=True)

        @pl.when(j == 0)
        def _():
            sum_scr[...] = su
            sq_scr[...] = sq

        @pl.when(j > 0)
        def _():
            sum_scr[...] += su
            sq_scr[...] += sq

    @pl.when(p == 1)
    def _():
        @pl.when(j == 0)
        def _():
            mean = sum_scr[...] * (1.0 / N)
            var = sq_scr[...] * (1.0 / N) - mean * mean
            inv = lax.rsqrt(var + 1e-5)
            scale = g_ref[...] * inv
            sc_scr[...] = scale
            sh_scr[...] = bt_ref[...] - mean * scale

        u = u_scr[pl.ds(j * BN, BN), :]
        y = jnp.maximum(u * sc_scr[...] + sh_scr[...], 0.0)
        nxlo_ref[...] = y[:, :HH] + xlo_ref[...]
        nxhi_ref[...] = y[:, HH:] + xhi_ref[...]


def _layer_tc(x_lo, x_hi, agg_lo, agg_hi, w1, b1, w2, b2, g, bt):
    node_spec = pl.BlockSpec((BN, HH), lambda p, j: (j, 0))
    full_spec = pl.BlockSpec((H, H), lambda p, j: (0, 0))
    row_spec = pl.BlockSpec((1, H), lambda p, j: (0, 0))
    return pl.pallas_call(
        _layer_tc_kernel,
        grid=(2, NB),
        in_specs=[node_spec, node_spec, node_spec, node_spec,
                  full_spec, row_spec, full_spec, row_spec,
                  row_spec, row_spec],
        out_specs=[node_spec, node_spec],
        out_shape=[
            jax.ShapeDtypeStruct((N, HH), jnp.float32),
            jax.ShapeDtypeStruct((N, HH), jnp.float32),
        ],
        scratch_shapes=[
            pltpu.VMEM((N, H), jnp.float32),
            pltpu.VMEM((1, H), jnp.float32),
            pltpu.VMEM((1, H), jnp.float32),
            pltpu.VMEM((1, H), jnp.float32),
            pltpu.VMEM((1, H), jnp.float32),
        ],
    )(x_lo, x_hi, agg_lo, agg_hi, w1, b1, w2, b2, g, bt)


def kernel(node_feat, edge_index, edge_feat, edge_W, edge_b, W1, b1, W2, b2,
           gamma, beta):
    pad = jnp.zeros((E_PAD - E,), jnp.int32)
    src = jnp.concatenate([edge_index[0].astype(jnp.int32), pad])
    dst = jnp.concatenate([edge_index[1].astype(jnp.int32), pad])
    ef_pad = jnp.concatenate(
        [edge_feat, jnp.zeros((E_PAD - E, EF), jnp.float32)])

    e_lo, e_hi = _edge_mlp(ef_pad, edge_W, edge_b.reshape(1, H))

    x_lo = node_feat[:, :HH]
    x_hi = node_feat[:, HH:]
    for i in range(W1.shape[0]):
        agg_lo, agg_hi = _sc_edge_phase(x_lo, x_hi, e_lo, e_hi, src, dst)
        x_lo, x_hi = _layer_tc(x_lo, x_hi, agg_lo, agg_hi,
                               W1[i], b1[i].reshape(1, H),
                               W2[i], b2[i].reshape(1, H),
                               gamma[i].reshape(1, H), beta[i].reshape(1, H))
    return jnp.concatenate([x_lo, x_hi], axis=1)
